# baseline (speedup 1.0000x reference)
import functools

import jax
import jax.numpy as jnp
from jax import lax
from jax.experimental import pallas as pl
from jax.experimental.pallas import tpu as pltpu
from jax.experimental.pallas import tpu_sc as plsc

N_CLS = 10000
N_CTX = 16
SUF = 60
SEQ = 77
D = 128
B = 1024
NV = 4

NW = 32
PER_TILE = NV * B // NW
C = 4
NCHUNK = PER_TILE // C
TILES_PER_V = NW // NV
IDX_ROWS = PER_TILE // C


def _sc_body(cls2d, pn, cn, sn, pp, cp, sp, pe, ce, se, ps, cs, ss,
             out, idx_v, pbuf, cbuf, sbuf, gs0, gs1, ws0, ws1):
  cid = lax.axis_index("c")
  sid = lax.axis_index("s")
  wid = sid * 2 + cid
  v = wid // TILES_PER_V
  r0 = (wid % TILES_PER_V) * IDX_ROWS
  i0_tile = wid * PER_TILE

  pltpu.sync_copy(cls2d.at[pl.ds(r0, IDX_ROWS)], idx_v)

  gsems = (gs0, gs1)
  wsems = (ws0, ws1)

  def run(pref, ctxt, suft):
    def start_gather(g, t):
      idx = idx_v.at[g]
      pltpu.async_copy(pref.at[idx], pbuf.at[t], gsems[t])
      pltpu.async_copy(ctxt.at[idx], cbuf.at[t], gsems[t])
      pltpu.async_copy(suft.at[idx], sbuf.at[t], gsems[t])

    def wait_gather(t):
      pltpu.make_async_copy(pref.at[pl.ds(0, C)], pbuf.at[t], gsems[t]).wait()
      pltpu.make_async_copy(ctxt.at[pl.ds(0, C)], cbuf.at[t], gsems[t]).wait()
      pltpu.make_async_copy(suft.at[pl.ds(0, C)], sbuf.at[t], gsems[t]).wait()

    def start_write(g, t):
      i0 = i0_tile + g * C
      wp = pltpu.async_copy(
          pbuf.at[t], out.at[pl.ds(i0, C), pl.ds(0, 1), :], wsems[t])
      wc = pltpu.async_copy(
          cbuf.at[t], out.at[pl.ds(i0, C), pl.ds(1, N_CTX), :], wsems[t])
      ws = pltpu.async_copy(
          sbuf.at[t], out.at[pl.ds(i0, C), pl.ds(1 + N_CTX, SUF), :], wsems[t])
      return wp, wc, ws

    start_gather(0, 0)
    start_gather(1, 1)

    def chunk(g, t):
      wait_gather(t)
      wp, wc, ws = start_write(g, t)

      @pl.when(g + 2 < NCHUNK)
      def _():
        wp.wait()
        wc.wait()
        ws.wait()
        start_gather(g + 2, t)

    def loop_body(gg, carry):
      chunk(gg * 2, 0)
      chunk(gg * 2 + 1, 1)
      return carry

    lax.fori_loop(0, NCHUNK // 2, loop_body, 0)

    for g, t in ((NCHUNK - 2, 0), (NCHUNK - 1, 1)):
      i0 = i0_tile + g * C
      pltpu.make_async_copy(
          pbuf.at[t], out.at[pl.ds(i0, C), pl.ds(0, 1), :], wsems[t]).wait()
      pltpu.make_async_copy(
          cbuf.at[t], out.at[pl.ds(i0, C), pl.ds(1, N_CTX), :], wsems[t]).wait()
      pltpu.make_async_copy(
          sbuf.at[t], out.at[pl.ds(i0, C), pl.ds(1 + N_CTX, SUF), :], wsems[t]).wait()

  @pl.when(v == 0)
  def _():
    run(pn, cn, sn)

  @pl.when(v == 1)
  def _():
    run(pp, cp, sp)

  @pl.when(v == 2)
  def _():
    run(pe, ce, se)

  @pl.when(v == 3)
  def _():
    run(ps, cs, ss)


_gather_call = functools.partial(
    pl.kernel,
    mesh=plsc.VectorSubcoreMesh(core_axis_name="c", subcore_axis_name="s"),
    out_type=jax.ShapeDtypeStruct((NV * B, SEQ, D), jnp.float32),
    scratch_types=[
        pltpu.VMEM((IDX_ROWS, C), jnp.int32),
        pltpu.VMEM((2, C, 1, D), jnp.float32),
        pltpu.VMEM((2, C, N_CTX, D), jnp.float32),
        pltpu.VMEM((2, C, SUF, D), jnp.float32),
        pltpu.SemaphoreType.DMA,
        pltpu.SemaphoreType.DMA,
        pltpu.SemaphoreType.DMA,
        pltpu.SemaphoreType.DMA,
    ],
    compiler_params=pltpu.CompilerParams(use_tc_tiling_on_sc=False),
)(_sc_body)


@jax.jit
def kernel(cls_id, ctx_pos, ctx_neg, ctx_evi, ctx_sub,
           prefix_pos, suffix_pos, prefix_neg, suffix_neg,
           prefix_evi, suffix_evi, prefix_sub, suffix_sub):
  cls2d = cls_id.astype(jnp.int32).reshape(B // C, C)
  return _gather_call(
      cls2d,
      prefix_neg, ctx_neg, suffix_neg,
      prefix_pos, ctx_pos, suffix_pos,
      prefix_evi, ctx_evi, suffix_evi,
      prefix_sub, ctx_sub, suffix_sub,
  )


# speedup vs baseline: 2.1457x; 2.1457x over previous
import functools

import jax
import jax.numpy as jnp
from jax import lax
from jax.experimental import pallas as pl
from jax.experimental.pallas import tpu as pltpu
from jax.experimental.pallas import tpu_sc as plsc

N_CLS = 10000
N_CTX = 16
SUF = 60
SEQ = 77
D = 128
B = 1024
NV = 4

NW = 32
PER_TILE = NV * B // NW
C = 4
NCHUNK = PER_TILE // C
TILES_PER_V = NW // NV
IDX_ROWS = PER_TILE // C


def _sc_body(cls2d, pn, cn, sn, pp, cp, sp, pe, ce, se, ps, cs, ss,
             out, idx_v, bbuf, gs0, gs1, ws0, ws1):
  cid = lax.axis_index("c")
  sid = lax.axis_index("s")
  wid = sid * 2 + cid
  v = wid // TILES_PER_V
  r0 = (wid % TILES_PER_V) * IDX_ROWS
  i0_tile = wid * PER_TILE

  pltpu.sync_copy(cls2d.at[pl.ds(r0, IDX_ROWS)], idx_v)

  gsems = (gs0, gs1)
  wsems = (ws0, ws1)

  def run(pref, ctxt, suft):
    def start_gather(g, t):
      idx = idx_v.at[g]
      pltpu.async_copy(pref.at[idx], bbuf.at[t, :, pl.ds(0, 1), :], gsems[t])
      pltpu.async_copy(ctxt.at[idx], bbuf.at[t, :, pl.ds(1, N_CTX), :], gsems[t])
      pltpu.async_copy(suft.at[idx], bbuf.at[t, :, pl.ds(1 + N_CTX, SUF), :], gsems[t])

    def wait_gather(t):
      pltpu.make_async_copy(
          pref.at[pl.ds(0, C)], bbuf.at[t, :, pl.ds(0, 1), :], gsems[t]).wait()
      pltpu.make_async_copy(
          ctxt.at[pl.ds(0, C)], bbuf.at[t, :, pl.ds(1, N_CTX), :], gsems[t]).wait()
      pltpu.make_async_copy(
          suft.at[pl.ds(0, C)], bbuf.at[t, :, pl.ds(1 + N_CTX, SUF), :], gsems[t]).wait()

    def start_write(g, t):
      i0 = i0_tile + g * C
      return pltpu.async_copy(
          bbuf.at[t, :, pl.ds(0, SEQ), :], out.at[pl.ds(i0, C)], wsems[t])

    start_gather(0, 0)
    start_gather(1, 1)

    def chunk(g, t):
      wait_gather(t)
      w = start_write(g, t)

      @pl.when(g + 2 < NCHUNK)
      def _():
        w.wait()
        start_gather(g + 2, t)

    def loop_body(gg, carry):
      chunk(gg * 2, 0)
      chunk(gg * 2 + 1, 1)
      return carry

    lax.fori_loop(0, NCHUNK // 2, loop_body, 0)

    for g, t in ((NCHUNK - 2, 0), (NCHUNK - 1, 1)):
      i0 = i0_tile + g * C
      pltpu.make_async_copy(
          bbuf.at[t, :, pl.ds(0, SEQ), :], out.at[pl.ds(i0, C)], wsems[t]).wait()

  @pl.when(v == 0)
  def _():
    run(pn, cn, sn)

  @pl.when(v == 1)
  def _():
    run(pp, cp, sp)

  @pl.when(v == 2)
  def _():
    run(pe, ce, se)

  @pl.when(v == 3)
  def _():
    run(ps, cs, ss)


_gather_call = functools.partial(
    pl.kernel,
    mesh=plsc.VectorSubcoreMesh(core_axis_name="c", subcore_axis_name="s"),
    out_type=jax.ShapeDtypeStruct((NV * B, SEQ, D), jnp.float32),
    scratch_types=[
        pltpu.VMEM((IDX_ROWS, C), jnp.int32),
        pltpu.VMEM((2, C, 80, D), jnp.float32),
        pltpu.SemaphoreType.DMA,
        pltpu.SemaphoreType.DMA,
        pltpu.SemaphoreType.DMA,
        pltpu.SemaphoreType.DMA,
    ],
)(_sc_body)


@jax.jit
def kernel(cls_id, ctx_pos, ctx_neg, ctx_evi, ctx_sub,
           prefix_pos, suffix_pos, prefix_neg, suffix_neg,
           prefix_evi, suffix_evi, prefix_sub, suffix_sub):
  cls2d = cls_id.astype(jnp.int32).reshape(B // C, C)
  return _gather_call(
      cls2d,
      prefix_neg, ctx_neg, suffix_neg,
      prefix_pos, ctx_pos, suffix_pos,
      prefix_evi, ctx_evi, suffix_evi,
      prefix_sub, ctx_sub, suffix_sub,
  )
